# baseline (speedup 1.0000x reference)
import jax
import jax.numpy as jnp
from jax import lax
from jax.experimental import pallas as pl
from jax.experimental.pallas import tpu as pltpu
from jax.experimental.pallas import tpu_sc as plsc

NUM_UNITS = 64
MAX_REL = 128
TABLE_ROWS = 2 * MAX_REL + 1
SEQ = 2048

NC = 2
NS = 16
NW = NC * NS
LANES = 16

CHUNK = 256
CPR = SEQ // CHUNK
ROWS_PER_W = SEQ // NW
N_ITERS = ROWS_PER_W * CPR
OUT_COLS = CHUNK // 2


def _chunk_coords(i):
    return i // CPR, pl.multiple_of((i % CPR) * CHUNK, CHUNK)


def _body(idx_hbm, table_hbm, out_hbm,
          table_v, idx0, idx1, rows0, rows1, is0, is1, os0, os1):
    wid = lax.axis_index("s") * NC + lax.axis_index("c")
    row_base = wid * ROWS_PER_W
    idx_bufs = (idx0, idx1)
    rows_bufs = (rows0, rows1)
    idx_sems = (is0, is1)
    out_sems = (os0, os1)

    pltpu.sync_copy(table_hbm, table_v)
    for b in range(2):
        r, jc = _chunk_coords(b)
        pltpu.async_copy(
            idx_hbm.at[row_base + r, pl.ds(jc, CHUNK)], idx_bufs[b],
            idx_sems[b])

    iota = lax.iota(jnp.int32, LANES)
    coloffs = [iota + d * LANES for d in range(NUM_UNITS // LANES)]
    dnums = lax.GatherDimensionNumbers(
        offset_dims=(), collapsed_slice_dims=(0,), start_index_map=(0,))
    lane_consts = [jnp.full((LANES, 1), j, jnp.int32) for j in range(LANES)]

    def step(g, carry):
        for b in range(2):
            i = 2 * g + b
            r, jc = _chunk_coords(i)
            row = row_base + r
            iv, rv = idx_bufs[b], rows_bufs[b]
            pltpu.make_async_copy(
                idx_hbm.at[row, pl.ds(jc, CHUNK)], iv, idx_sems[b]).wait()
            @pl.when(g >= 1)
            def _():
                pltpu.make_async_copy(
                    rv, out_hbm.at[row, pl.ds(pl.multiple_of(jc // 2, OUT_COLS), OUT_COLS)],
                    out_sems[b]).wait()

            @plsc.parallel_loop(0, CHUNK // LANES, 1, unroll=2)
            def _grp(gg):
                p0 = gg * LANES
                rb_vec = iv[pl.ds(p0, LANES)] + MAX_REL
                rb_vec = jnp.minimum(
                    jnp.maximum(rb_vec, 0), TABLE_ROWS - 1) * NUM_UNITS
                for j in range(LANES):
                    rbj = lax.gather(
                        rb_vec, lane_consts[j], dnums, slice_sizes=(1,),
                        mode=lax.GatherScatterMode.PROMISE_IN_BOUNDS)
                    for d in range(NUM_UNITS // LANES):
                        val = plsc.load_gather(table_v, [rbj + coloffs[d]])
                        rv[p0 // 2 + j // 2,
                           pl.ds((j % 2) * NUM_UNITS + d * LANES,
                                 LANES)] = val
            pltpu.async_copy(
                rv, out_hbm.at[row, pl.ds(pl.multiple_of(jc // 2, OUT_COLS), OUT_COLS)], out_sems[b])
            ip = jnp.minimum(i + 2, N_ITERS - 1)
            rp_, jp = _chunk_coords(ip)
            pltpu.async_copy(
                idx_hbm.at[row_base + rp_, pl.ds(jp, CHUNK)], iv,
                idx_sems[b])
        return carry

    lax.fori_loop(0, N_ITERS // 2, step, 0)

    for b in range(2):
        pltpu.make_async_copy(
            idx_hbm.at[row_base, pl.ds(0, CHUNK)], idx_bufs[b],
            idx_sems[b]).wait()
        pltpu.make_async_copy(
            rows_bufs[b], out_hbm.at[row_base, pl.ds(0, OUT_COLS)],
            out_sems[b]).wait()


@jax.jit
def _run(idx, table_flat):
    mesh = plsc.VectorSubcoreMesh(
        core_axis_name="c", subcore_axis_name="s", num_cores=NC,
        num_subcores=NS)
    return pl.kernel(
        _body,
        out_type=jax.ShapeDtypeStruct(
            (SEQ, SEQ // 2, 2 * NUM_UNITS), jnp.float32),
        mesh=mesh,
        scratch_types=[
            pltpu.VMEM((TABLE_ROWS * NUM_UNITS,), jnp.float32),
            pltpu.VMEM((CHUNK,), jnp.int32),
            pltpu.VMEM((CHUNK,), jnp.int32),
            pltpu.VMEM((OUT_COLS, 2 * NUM_UNITS), jnp.float32),
            pltpu.VMEM((OUT_COLS, 2 * NUM_UNITS), jnp.float32),
            pltpu.SemaphoreType.DMA,
            pltpu.SemaphoreType.DMA,
            pltpu.SemaphoreType.DMA,
            pltpu.SemaphoreType.DMA,
        ],
        compiler_params=pltpu.CompilerParams(
            use_tc_tiling_on_sc=True, needs_layout_passes=False),
    )(idx, table_flat)


def kernel(relative_positions, embeddings_table):
    idx = relative_positions.astype(jnp.int32)
    out3 = _run(idx, embeddings_table.reshape(TABLE_ROWS * NUM_UNITS))
    return out3.reshape(SEQ, SEQ, NUM_UNITS)


# speedup vs baseline: 1.0997x; 1.0997x over previous
import jax
import jax.numpy as jnp
from jax import lax
from jax.experimental import pallas as pl
from jax.experimental.pallas import tpu as pltpu
from jax.experimental.pallas import tpu_sc as plsc

NUM_UNITS = 64
MAX_REL = 128
TABLE_ROWS = 2 * MAX_REL + 1
SEQ = 2048

NC = 2
NS = 16
NW = NC * NS
LANES = 16

CHUNK = 256
CPR = SEQ // CHUNK
ROWS_PER_W = SEQ // NW
N_ITERS = ROWS_PER_W * CPR


def _chunk_coords(i):
    return i // CPR, (i % CPR) * CHUNK


def _body(idx_hbm, table_hbm, out_hbm,
          table_v, idx0, idx1, rows0, rows1, is0, is1, os0, os1):
    wid = lax.axis_index("s") * NC + lax.axis_index("c")
    row_base = wid * ROWS_PER_W
    idx_bufs = (idx0, idx1)
    rows_bufs = (rows0, rows1)
    idx_sems = (is0, is1)
    out_sems = (os0, os1)

    pltpu.sync_copy(table_hbm, table_v)
    for b in range(2):
        r, jc = _chunk_coords(b)
        pltpu.async_copy(
            idx_hbm.at[row_base + r, pl.ds(jc, CHUNK)], idx_bufs[b],
            idx_sems[b])

    iota = lax.iota(jnp.int32, LANES)
    coloffs = [iota + d * LANES for d in range(NUM_UNITS // LANES)]
    dnums = lax.GatherDimensionNumbers(
        offset_dims=(), collapsed_slice_dims=(0,), start_index_map=(0,))
    lane_consts = [jnp.full((LANES, 1), j, jnp.int32) for j in range(LANES)]

    def step(g, carry):
        for b in range(2):
            i = 2 * g + b
            r, jc = _chunk_coords(i)
            row = row_base + r
            iv, rv = idx_bufs[b], rows_bufs[b]
            pltpu.make_async_copy(
                idx_hbm.at[row, pl.ds(jc, CHUNK)], iv, idx_sems[b]).wait()
            @pl.when(g >= 1)
            def _():
                pltpu.make_async_copy(
                    rv, out_hbm.at[row, pl.ds(jc, CHUNK)],
                    out_sems[b]).wait()

            @plsc.parallel_loop(0, CHUNK // LANES, 1, unroll=2)
            def _grp(gg):
                p0 = gg * LANES
                rb_vec = iv[pl.ds(p0, LANES)] + MAX_REL
                rb_vec = jnp.minimum(
                    jnp.maximum(rb_vec, 0), TABLE_ROWS - 1) * NUM_UNITS
                for j in range(LANES):
                    rbj = lax.gather(
                        rb_vec, lane_consts[j], dnums, slice_sizes=(1,),
                        mode=lax.GatherScatterMode.PROMISE_IN_BOUNDS)
                    for d in range(NUM_UNITS // LANES):
                        val = plsc.load_gather(table_v, [rbj + coloffs[d]])
                        rv[p0 + j, pl.ds(d * LANES, LANES)] = val
            pltpu.async_copy(
                rv, out_hbm.at[row, pl.ds(jc, CHUNK)], out_sems[b])
            ip = jnp.minimum(i + 2, N_ITERS - 1)
            rp_, jp = _chunk_coords(ip)
            pltpu.async_copy(
                idx_hbm.at[row_base + rp_, pl.ds(jp, CHUNK)], iv,
                idx_sems[b])
        return carry

    lax.fori_loop(0, N_ITERS // 2, step, 0)

    for b in range(2):
        pltpu.make_async_copy(
            idx_hbm.at[row_base, pl.ds(0, CHUNK)], idx_bufs[b],
            idx_sems[b]).wait()
        pltpu.make_async_copy(
            rows_bufs[b], out_hbm.at[row_base, pl.ds(0, CHUNK)],
            out_sems[b]).wait()


@jax.jit
def _run(idx, table_flat):
    mesh = plsc.VectorSubcoreMesh(
        core_axis_name="c", subcore_axis_name="s", num_cores=NC,
        num_subcores=NS)
    return pl.kernel(
        _body,
        out_type=jax.ShapeDtypeStruct((SEQ, SEQ, NUM_UNITS), jnp.float32),
        mesh=mesh,
        scratch_types=[
            pltpu.VMEM((TABLE_ROWS * NUM_UNITS,), jnp.float32),
            pltpu.VMEM((CHUNK,), jnp.int32),
            pltpu.VMEM((CHUNK,), jnp.int32),
            pltpu.VMEM((CHUNK, NUM_UNITS), jnp.float32),
            pltpu.VMEM((CHUNK, NUM_UNITS), jnp.float32),
            pltpu.SemaphoreType.DMA,
            pltpu.SemaphoreType.DMA,
            pltpu.SemaphoreType.DMA,
            pltpu.SemaphoreType.DMA,
        ],
        compiler_params=pltpu.CompilerParams(
            use_tc_tiling_on_sc=True, needs_layout_passes=False),
    )(idx, table_flat)


def kernel(relative_positions, embeddings_table):
    idx = relative_positions.astype(jnp.int32)
    return _run(idx, embeddings_table.reshape(TABLE_ROWS * NUM_UNITS))
